# baseline (speedup 1.0000x reference)
import jax
import jax.numpy as jnp
from jax.experimental import pallas as pl
from jax.experimental.pallas import tpu as pltpu

NV_PREV = 10242
CHUNKS = 18
CH = NV_PREV // CHUNKS
NBUF = 6
LOOKAHEAD = 2
assert CH * CHUNKS == NV_PREV


def _copy_body(x_ref, o_ref, buf, in_sems, out_sems):
    xr = x_ref.reshape(x_ref.shape[0], 16, 64)
    orr = o_ref.reshape(NV_PREV, 16, 64)

    def in_cp(g):
        return pltpu.make_async_copy(
            xr.at[pl.ds(g * CH, CH)], buf.at[g % NBUF], in_sems.at[g % NBUF])

    def out_cp(g):
        return pltpu.make_async_copy(
            buf.at[g % NBUF], orr.at[pl.ds(g * CH, CH)], out_sems.at[g % NBUF])

    for g in range(LOOKAHEAD):
        in_cp(g).start()
    for g in range(CHUNKS):
        in_cp(g).wait()
        out_cp(g).start()
        nk = g + LOOKAHEAD
        if nk < CHUNKS:
            if nk >= NBUF:
                out_cp(nk - NBUF).wait()
            in_cp(nk).start()
    for g in range(CHUNKS - NBUF, CHUNKS):
        out_cp(g).wait()


def kernel(x):
    n, a, b, c = x.shape
    return pl.pallas_call(
        _copy_body,
        out_shape=jax.ShapeDtypeStruct((NV_PREV, a, b, c), x.dtype),
        in_specs=[pl.BlockSpec(memory_space=pl.ANY)],
        out_specs=pl.BlockSpec(memory_space=pl.ANY),
        scratch_shapes=[
            pltpu.VMEM((NBUF, CH, 16, 64), x.dtype),
            pltpu.SemaphoreType.DMA((NBUF,)),
            pltpu.SemaphoreType.DMA((NBUF,)),
        ],
    )(x)
